# baseline (speedup 1.0000x reference)
import functools

import jax
import jax.numpy as jnp
from jax import lax
from jax.experimental import pallas as pl
from jax.experimental.pallas import tpu as pltpu
from jax.experimental.pallas import tpu_sc as plsc

K_NEURONS = 10000
FEAT = 784
BATCH = 4096

BB = 512
BK = 1024
NB = BATCH // BB
NK = -(-K_NEURONS // BK)

BIG = 3.0e38

FEAT_PAD = 896

SC_NC = 2
SC_NS = 16
SC_NW = SC_NC * SC_NS
ROWS_PER_W = BATCH // SC_NW


def _dist_argmin_body(x_ref, w_ref, idx_ref, best_ref, bidx_ref):
    j = pl.program_id(1)

    @pl.when(j == 0)
    def _init():
        best_ref[...] = jnp.full((BB, 1), BIG, jnp.float32)
        bidx_ref[...] = jnp.zeros((BB, 1), jnp.int32)

    x = x_ref[...]
    w = w_ref[...]
    cross = lax.dot_general(x, w, (((1,), (1,)), ((), ())),
                            preferred_element_type=jnp.float32)
    x_sq = jnp.sum(x * x, axis=1, keepdims=True)
    w_sq = jnp.sum(w * w, axis=1)[None, :]
    dist = x_sq - 2.0 * cross + w_sq
    col = lax.broadcasted_iota(jnp.int32, (BB, BK), 1) + j * BK
    dist = jnp.where(col < K_NEURONS, dist, BIG)
    m = jnp.min(dist, axis=1, keepdims=True)
    am = jnp.min(jnp.where(dist == m, col, jnp.int32(2**31 - 1)),
                 axis=1, keepdims=True)
    upd = m < best_ref[...]
    best_ref[...] = jnp.where(upd, m, best_ref[...])
    bidx_ref[...] = jnp.where(upd, am, bidx_ref[...])

    @pl.when(j == NK - 1)
    def _emit():
        idx_ref[...] = bidx_ref[...]


def _bmu_indices(x, weights):
    return pl.pallas_call(
        _dist_argmin_body,
        grid=(NB, NK),
        in_specs=[
            pl.BlockSpec((BB, FEAT), lambda i, j: (i, 0)),
            pl.BlockSpec((BK, FEAT), lambda i, j: (j, 0)),
        ],
        out_specs=pl.BlockSpec((BB, 1), lambda i, j: (i, 0)),
        out_shape=jax.ShapeDtypeStruct((BATCH, 1), jnp.int32),
        scratch_shapes=[
            pltpu.VMEM((BB, 1), jnp.float32),
            pltpu.VMEM((BB, 1), jnp.int32),
        ],
        compiler_params=pltpu.CompilerParams(
            dimension_semantics=("parallel", "arbitrary")),
    )(x, weights)


def _gather_body(table_hbm, idx_hbm, out_hbm, idx_v, rows_v, sem):
    wid = lax.axis_index("s") * SC_NC + lax.axis_index("c")
    base = wid * ROWS_PER_W
    pltpu.sync_copy(idx_hbm.at[pl.ds(base, ROWS_PER_W)], idx_v)
    pltpu.async_copy(table_hbm.at[idx_v], rows_v, sem).wait()
    pltpu.sync_copy(rows_v, out_hbm.at[pl.ds(base, ROWS_PER_W)])


def _gather_rows(table, idx):
    mesh = plsc.VectorSubcoreMesh(core_axis_name="c", subcore_axis_name="s")
    return pl.kernel(
        _gather_body,
        out_type=jax.ShapeDtypeStruct((BATCH, FEAT_PAD), jnp.float32),
        mesh=mesh,
        scratch_types=[
            pltpu.VMEM((ROWS_PER_W,), jnp.int32),
            pltpu.VMEM((ROWS_PER_W, FEAT_PAD), jnp.float32),
            pltpu.SemaphoreType.DMA,
        ],
    )(table, idx)


def kernel(inputs, weights):
    x = inputs.reshape(-1, FEAT)
    idx = _bmu_indices(x, weights).reshape(BATCH)
    table = jnp.pad(weights, ((0, 0), (0, FEAT_PAD - FEAT)))
    return _gather_rows(table, idx)[:, :FEAT]


# speedup vs baseline: 1.0712x; 1.0712x over previous
import functools

import jax
import jax.numpy as jnp
from jax import lax
from jax.experimental import pallas as pl
from jax.experimental.pallas import tpu as pltpu
from jax.experimental.pallas import tpu_sc as plsc

K_NEURONS = 10000
FEAT = 784
BATCH = 4096

BB = 512
BK = 2000
NB = BATCH // BB
NK = K_NEURONS // BK

BIG = 3.0e38

FEAT_PAD = 896

SC_NC = 2
SC_NS = 16
SC_NW = SC_NC * SC_NS
ROWS_PER_W = BATCH // SC_NW


def _dist_argmin_body(x_ref, w_ref, idx_ref,
                      x2_ref, xsq_ref, wsq_ref, best_ref, bidx_ref):
    i = pl.program_id(0)
    j = pl.program_id(1)

    @pl.when(j == 0)
    def _init():
        x = x_ref[...]
        x2_ref[...] = x * -2.0
        xsq_ref[...] = jnp.sum(x * x, axis=1, keepdims=True)
        best_ref[...] = jnp.full((BB, 1), BIG, jnp.float32)
        bidx_ref[...] = jnp.zeros((BB, 1), jnp.int32)

    @pl.when(i == 0)
    def _wsq():
        w = w_ref[...]
        wsq_ref[j, :] = jnp.sum(w * w, axis=1)

    cross2 = lax.dot_general(x2_ref[...], w_ref[...], (((1,), (1,)), ((), ())),
                             preferred_element_type=jnp.float32)
    dist = (xsq_ref[...] + cross2) + wsq_ref[j, :][None, :]
    col = lax.broadcasted_iota(jnp.int32, (BB, BK), 1) + j * BK
    m = jnp.min(dist, axis=1, keepdims=True)
    am = jnp.min(jnp.where(dist == m, col, jnp.int32(2**31 - 1)),
                 axis=1, keepdims=True)
    upd = m < best_ref[...]
    best_ref[...] = jnp.where(upd, m, best_ref[...])
    bidx_ref[...] = jnp.where(upd, am, bidx_ref[...])

    @pl.when(j == NK - 1)
    def _emit():
        idx_ref[...] = bidx_ref[...]


def _bmu_indices(x, weights):
    return pl.pallas_call(
        _dist_argmin_body,
        grid=(NB, NK),
        in_specs=[
            pl.BlockSpec((BB, FEAT), lambda i, j: (i, 0)),
            pl.BlockSpec((BK, FEAT), lambda i, j: (j, 0)),
        ],
        out_specs=pl.BlockSpec((BB, 1), lambda i, j: (i, 0)),
        out_shape=jax.ShapeDtypeStruct((BATCH, 1), jnp.int32),
        scratch_shapes=[
            pltpu.VMEM((BB, FEAT), jnp.float32),
            pltpu.VMEM((BB, 1), jnp.float32),
            pltpu.VMEM((NK, BK), jnp.float32),
            pltpu.VMEM((BB, 1), jnp.float32),
            pltpu.VMEM((BB, 1), jnp.int32),
        ],
        compiler_params=pltpu.CompilerParams(
            dimension_semantics=("parallel", "arbitrary")),
    )(x, weights)


def _gather_body(table_hbm, idx_hbm, out_hbm, idx_v, rows_v, sem):
    wid = lax.axis_index("s") * SC_NC + lax.axis_index("c")
    base = wid * ROWS_PER_W
    pltpu.sync_copy(idx_hbm.at[pl.ds(base, ROWS_PER_W)], idx_v)
    pltpu.async_copy(table_hbm.at[idx_v], rows_v, sem).wait()
    pltpu.sync_copy(rows_v, out_hbm.at[pl.ds(base, ROWS_PER_W)])


def _gather_rows(table, idx):
    mesh = plsc.VectorSubcoreMesh(core_axis_name="c", subcore_axis_name="s")
    return pl.kernel(
        _gather_body,
        out_type=jax.ShapeDtypeStruct((BATCH, FEAT_PAD), jnp.float32),
        mesh=mesh,
        scratch_types=[
            pltpu.VMEM((ROWS_PER_W,), jnp.int32),
            pltpu.VMEM((ROWS_PER_W, FEAT_PAD), jnp.float32),
            pltpu.SemaphoreType.DMA,
        ],
    )(table, idx)


def kernel(inputs, weights):
    x = inputs.reshape(-1, FEAT)
    idx = _bmu_indices(x, weights).reshape(BATCH)
    table = jnp.pad(weights, ((0, 0), (0, FEAT_PAD - FEAT)))
    return _gather_rows(table, idx)[:, :FEAT]
